# Initial kernel scaffold; baseline (speedup 1.0000x reference)
import jax
import jax.numpy as jnp
from jax.experimental import pallas as pl


def kernel(feat, edge_index, labels, times, P, W, b):
    raise NotImplementedError("write your pallas kernel here")


# speedup vs baseline: 1.1522x; 1.1522x over previous
import functools

import jax
import jax.numpy as jnp
from jax import lax
from jax.experimental import pallas as pl
from jax.experimental.pallas import tpu as pltpu
from jax.experimental.pallas import tpu_sc as plsc

_N = 10000
_E = 320000
_D = 128
_NL = 4
_NT = 8
_SPLIT = 6

_NC, _NS = 2, 16
_NW = _NC * _NS
_EPW = _E // _NW
_KW = 80
_KA = 80
_NPAD = 10240
_RPW = _NPAD // _NS

_R = 256
_GRID = _NPAD // _R


def _sc_mesh():
    return plsc.VectorSubcoreMesh(core_axis_name="c", subcore_axis_name="s")


def _sc_deg(dst):
    @functools.partial(
        pl.kernel,
        mesh=_sc_mesh(),
        out_type=jax.ShapeDtypeStruct((_NC * _NPAD,), jnp.float32),
        scratch_types=[
            pltpu.VMEM((_KW,), jnp.int32),
            pltpu.VMEM((_KW,), jnp.float32),
            pltpu.VMEM((_RPW,), jnp.float32),
            pltpu.VMEM_SHARED((_NPAD,), jnp.float32),
        ],
    )
    def k(dst_hbm, out_hbm, idx_v, ones_v, z_v, deg_sh):
        cid = lax.axis_index("c")
        sid = lax.axis_index("s")
        base = (cid * _NS + sid) * _EPW

        @pl.loop(0, _KW, step=16)
        def _(i):
            ones_v[pl.ds(i, 16)] = jnp.full((16,), 1.0, jnp.float32)

        @pl.loop(0, _RPW, step=16)
        def _(i):
            z_v[pl.ds(i, 16)] = jnp.zeros((16,), jnp.float32)

        pltpu.sync_copy(z_v, deg_sh.at[pl.ds(sid * _RPW, _RPW)])
        plsc.subcore_barrier()

        @pl.loop(0, _EPW, step=_KW)
        def _(j):
            pltpu.sync_copy(dst_hbm.at[pl.ds(base + j, _KW)], idx_v)
            pltpu.sync_copy(ones_v, deg_sh.at[idx_v], add=True)

        plsc.subcore_barrier()
        pltpu.sync_copy(deg_sh.at[pl.ds(sid * _RPW, _RPW)], z_v)
        pltpu.sync_copy(z_v, out_hbm.at[pl.ds(cid * _NPAD + sid * _RPW, _RPW)])

    return k(dst).reshape(_NC, _NPAD)


def _sc_agg(h, src, dst):
    @functools.partial(
        pl.kernel,
        mesh=_sc_mesh(),
        out_type=jax.ShapeDtypeStruct((_NC, _NPAD, _D), jnp.float32),
        scratch_types=[
            pltpu.VMEM((_KA,), jnp.int32),
            pltpu.VMEM((_KA,), jnp.int32),
            pltpu.VMEM((_KA, _D), jnp.float32),
            pltpu.VMEM((8, _D), jnp.float32),
            pltpu.VMEM_SHARED((_NPAD, _D), jnp.float32),
            pltpu.SemaphoreType.DMA,
        ],
    )
    def k(h_hbm, src_hbm, dst_hbm, out_hbm, sidx_v, didx_v, rows_v, z_v,
          agg_sh, sem):
        cid = lax.axis_index("c")
        sid = lax.axis_index("s")
        base = (cid * _NS + sid) * _EPW

        @pl.loop(0, 8)
        def _(r):
            @pl.loop(0, _D, step=16)
            def _(i):
                z_v[r, pl.ds(i, 16)] = jnp.zeros((16,), jnp.float32)

        @pl.loop(0, _RPW, step=8)
        def _(r):
            pltpu.sync_copy(z_v, agg_sh.at[pl.ds(sid * _RPW + r, 8)])

        plsc.subcore_barrier()

        @pl.loop(0, _EPW, step=_KA)
        def _(j):
            pltpu.sync_copy(src_hbm.at[pl.ds(base + j, _KA)], sidx_v)
            pltpu.async_copy(h_hbm.at[sidx_v], rows_v, sem).wait()
            pltpu.sync_copy(dst_hbm.at[pl.ds(base + j, _KA)], didx_v)
            pltpu.sync_copy(rows_v, agg_sh.at[didx_v], add=True)

        plsc.subcore_barrier()
        pltpu.sync_copy(agg_sh.at[pl.ds(sid * _RPW, _RPW)],
                        out_hbm.at[cid, pl.ds(sid * _RPW, _RPW)])

    return k(h, src, dst)


def _tc_combine(aggp, lab3d, tim3d):
    def body(aggp_ref, lab_ref, tim_ref, agg_ref, gs_ref, gc_ref):
        i = pl.program_id(0)
        a = aggp_ref[0] + aggp_ref[1]
        agg_ref[...] = a

        @pl.when(i == 0)
        def _():
            gs_ref[...] = jnp.zeros_like(gs_ref)
            gc_ref[...] = jnp.zeros_like(gc_ref)

        g = tim_ref[0, 0, :] * _NL + lab_ref[0, 0, :]
        onehot = (lax.broadcasted_iota(jnp.int32, (_NL * _NT, _R), 0)
                  == g[None, :]).astype(jnp.float32)
        gs_ref[...] += lax.dot_general(onehot, a, (((1,), (0,)), ((), ())),
                                       preferred_element_type=jnp.float32)
        gc_ref[...] += jnp.sum(onehot, axis=1)[:, None]

    return pl.pallas_call(
        body,
        grid=(_GRID,),
        in_specs=[
            pl.BlockSpec((_NC, _R, _D), lambda i: (0, i, 0)),
            pl.BlockSpec((1, 1, _R), lambda i: (i, 0, 0)),
            pl.BlockSpec((1, 1, _R), lambda i: (i, 0, 0)),
        ],
        out_specs=[
            pl.BlockSpec((_R, _D), lambda i: (i, 0)),
            pl.BlockSpec((_NL * _NT, _D), lambda i: (0, 0)),
            pl.BlockSpec((_NL * _NT, _D), lambda i: (0, 0)),
        ],
        out_shape=[
            jax.ShapeDtypeStruct((_NPAD, _D), jnp.float32),
            jax.ShapeDtypeStruct((_NL * _NT, _D), jnp.float32),
            jax.ShapeDtypeStruct((_NL * _NT, _D), jnp.float32),
        ],
    )(aggp, lab3d, tim3d)


def _tc_transform(agg, lab3d, tim3d, norm, t_all, c_all):
    ng = _NL * _SPLIT

    def body(agg_ref, lab_ref, tim_ref, norm_ref, t_ref, c_ref, out_ref,
             cs_ref):
        i = pl.program_id(0)
        x = agg_ref[...]
        lab = lab_ref[0, 0, :]
        tim = tim_ref[0, 0, :]
        train = (tim < _SPLIT) & (lab >= 0)
        gid = jnp.where(train, lab * _SPLIT + tim, ng)
        acc = x * (~train).astype(jnp.float32)[:, None]
        for g in range(ng):
            m = (gid == g).astype(jnp.float32)[:, None]
            xm = x * m
            acc += lax.dot_general(xm, t_ref[g], (((1,), (1,)), ((), ())),
                                   preferred_element_type=jnp.float32)
            acc += m * c_ref[g][None, :]
        out = acc * norm_ref[...]
        out_ref[...] = out

        @pl.when(i == 0)
        def _():
            cs_ref[...] = jnp.zeros_like(cs_ref)

        cs_ref[0, :] += jnp.sum(out, axis=0)
        cs_ref[1, :] += jnp.sum(out * out, axis=0)

    return pl.pallas_call(
        body,
        grid=(_GRID,),
        in_specs=[
            pl.BlockSpec((_R, _D), lambda i: (i, 0)),
            pl.BlockSpec((1, 1, _R), lambda i: (i, 0, 0)),
            pl.BlockSpec((1, 1, _R), lambda i: (i, 0, 0)),
            pl.BlockSpec((_R, 1), lambda i: (i, 0)),
            pl.BlockSpec((ng, _D, _D), lambda i: (0, 0, 0)),
            pl.BlockSpec((ng, _D), lambda i: (0, 0)),
        ],
        out_specs=[
            pl.BlockSpec((_R, _D), lambda i: (i, 0)),
            pl.BlockSpec((2, _D), lambda i: (0, 0)),
        ],
        out_shape=[
            jax.ShapeDtypeStruct((_NPAD, _D), jnp.float32),
            jax.ShapeDtypeStruct((2, _D), jnp.float32),
        ],
    )(agg, lab3d, tim3d, norm, t_all, c_all)


def _tc_final(outv, ws, cvec):
    def body(o_ref, w_ref, c_ref, f_ref):
        f_ref[...] = lax.dot_general(
            o_ref[...], w_ref[...], (((1,), (1,)), ((), ())),
            preferred_element_type=jnp.float32) + c_ref[0][None, :]

    return pl.pallas_call(
        body,
        grid=(_GRID,),
        in_specs=[
            pl.BlockSpec((_R, _D), lambda i: (i, 0)),
            pl.BlockSpec((_D, _D), lambda i: (0, 0)),
            pl.BlockSpec((1, _D), lambda i: (0, 0)),
        ],
        out_specs=pl.BlockSpec((_R, _D), lambda i: (i, 0)),
        out_shape=jax.ShapeDtypeStruct((_NPAD, _D), jnp.float32),
    )(outv, ws, cvec)


def kernel(feat, edge_index, labels, times, P, W, b):
    src = edge_index[0]
    dst = edge_index[1]
    pad = _NPAD - _N
    lab_p = jnp.pad(labels, (0, pad), constant_values=-1)
    tim_p = jnp.pad(times, (0, pad), constant_values=127)
    lab3d = lab_p.reshape(_GRID, 1, _R)
    tim3d = tim_p.reshape(_GRID, 1, _R)

    degp = _sc_deg(dst)

    deg = jnp.clip(degp[0, :_N] + degp[1, :_N], 1.0, None)
    norm = (deg ** -0.5)[:, None]
    h = feat * norm
    prev_cov = []
    for y in range(_NL):
        m = (labels == y).astype(feat.dtype)[:, None]
        cnt = jnp.sum(m)
        mu = jnp.sum(h * m, axis=0) / cnt
        cen = (h - mu[None, :]) * m
        prev_cov.append(cen.T @ cen / (cnt - 1.0))

    cur = [[None] * _NT for _ in range(_NL)]
    for y1 in range(_NL):
        for t1 in range(_NT):
            denom = jnp.asarray(0.0, feat.dtype)
            for y2 in range(_NL):
                for t2 in range(_NT):
                    c = 2.0 if abs(t2 - t1) > min(_NT - 1 - t1, t1) else 1.0
                    denom = denom + P[y1, t1, y2, t2] * c
            denom = denom * denom
            acc = jnp.zeros((_D, _D), feat.dtype)
            for y2 in range(_NL):
                temp = jnp.asarray(0.0, feat.dtype)
                for t2 in range(_NT):
                    c = 4.0 if abs(t2 - t1) > min(_NT - 1 - t1, t1) else 1.0
                    temp = temp + P[y1, t1, y2, t2] * c
                temp = temp / denom
                acc = acc + temp * prev_cov[y2]
            cur[y1][t1] = acc

    trans = [[None] * _SPLIT for _ in range(_NL)]
    for y1 in range(_NL):
        Lm, Qm = jnp.linalg.eigh(cur[y1][_NT - 1])
        A = Qm @ jnp.diag(jnp.sqrt(Lm))
        for t1 in range(_SPLIT):
            Lv, Q = jnp.linalg.eigh(cur[y1][t1])
            trans[y1][t1] = A @ jnp.diag(1.0 / jnp.sqrt(Lv)) @ Q.T
    t_all = jnp.stack([trans[y][t] for y in range(_NL)
                       for t in range(_SPLIT)])

    h_p = jnp.pad(h, ((0, pad), (0, 0)))
    norm_p = jnp.pad(norm, ((0, pad), (0, 0)), constant_values=1.0)
    aggp = _sc_agg(h_p, src, dst)
    agg, gsum, gcnt = _tc_combine(aggp, lab3d, tim3d)

    mean = gsum / jnp.maximum(1.0, gcnt)
    mu_all = jnp.transpose(mean.reshape(_NT, _NL, _D),
                           (1, 0, 2))[:, :_SPLIT].reshape(_NL * _SPLIT, _D)
    c_all = mu_all - jnp.einsum('gd,ged->ge', mu_all, t_all)

    outv, cs = _tc_transform(agg, lab3d, tim3d, norm_p, t_all, c_all)


    m = cs[0] / float(_N)
    var = (cs[1] - float(_N) * m * m) / float(_N - 1)
    s = jnp.sqrt(var)
    ws = W / s[None, :]
    cvec = (b - (m / s) @ W.T)[None, :]

    final = _tc_final(outv, ws, cvec)
    return final[:_N]


# speedup vs baseline: 1.1567x; 1.0039x over previous
import functools

import jax
import jax.numpy as jnp
from jax import lax
from jax.experimental import pallas as pl
from jax.experimental.pallas import tpu as pltpu
from jax.experimental.pallas import tpu_sc as plsc

_N = 10000
_E = 320000
_D = 128
_NL = 4
_NT = 8
_SPLIT = 6

_NC, _NS = 2, 16
_NW = _NC * _NS
_EPW = _E // _NW
_KW = 80
_KA = 80
_NPAD = 10240
_RPW = _NPAD // _NS

_R = 256
_GRID = _NPAD // _R


def _sc_mesh():
    return plsc.VectorSubcoreMesh(core_axis_name="c", subcore_axis_name="s")


def _sc_deg(dst):
    @functools.partial(
        pl.kernel,
        mesh=_sc_mesh(),
        out_type=jax.ShapeDtypeStruct((_NC * _NPAD,), jnp.float32),
        scratch_types=[
            pltpu.VMEM((_KW,), jnp.int32),
            pltpu.VMEM((_KW,), jnp.float32),
            pltpu.VMEM((_RPW,), jnp.float32),
            pltpu.VMEM_SHARED((_NPAD,), jnp.float32),
        ],
    )
    def k(dst_hbm, out_hbm, idx_v, ones_v, z_v, deg_sh):
        cid = lax.axis_index("c")
        sid = lax.axis_index("s")
        base = (cid * _NS + sid) * _EPW

        @pl.loop(0, _KW, step=16)
        def _(i):
            ones_v[pl.ds(i, 16)] = jnp.full((16,), 1.0, jnp.float32)

        @pl.loop(0, _RPW, step=16)
        def _(i):
            z_v[pl.ds(i, 16)] = jnp.zeros((16,), jnp.float32)

        pltpu.sync_copy(z_v, deg_sh.at[pl.ds(sid * _RPW, _RPW)])
        plsc.subcore_barrier()

        @pl.loop(0, _EPW, step=_KW)
        def _(j):
            pltpu.sync_copy(dst_hbm.at[pl.ds(base + j, _KW)], idx_v)
            pltpu.sync_copy(ones_v, deg_sh.at[idx_v], add=True)

        plsc.subcore_barrier()
        pltpu.sync_copy(deg_sh.at[pl.ds(sid * _RPW, _RPW)], z_v)
        pltpu.sync_copy(z_v, out_hbm.at[pl.ds(cid * _NPAD + sid * _RPW, _RPW)])

    return k(dst).reshape(_NC, _NPAD)


def _sc_agg(h, src, dst):
    @functools.partial(
        pl.kernel,
        mesh=_sc_mesh(),
        out_type=jax.ShapeDtypeStruct((_NC, _NPAD, _D), jnp.float32),
        scratch_types=[
            pltpu.VMEM((_KA,), jnp.int32),
            pltpu.VMEM((_KA,), jnp.int32),
            pltpu.VMEM((_KA, _D), jnp.float32),
            pltpu.VMEM((8, _D), jnp.float32),
            pltpu.VMEM_SHARED((_NPAD, _D), jnp.float32),
            pltpu.SemaphoreType.DMA,
        ],
    )
    def k(h_hbm, src_hbm, dst_hbm, out_hbm, sidx_v, didx_v, rows_v, z_v,
          agg_sh, sem):
        cid = lax.axis_index("c")
        sid = lax.axis_index("s")
        base = (cid * _NS + sid) * _EPW

        @pl.loop(0, 8)
        def _(r):
            @pl.loop(0, _D, step=16)
            def _(i):
                z_v[r, pl.ds(i, 16)] = jnp.zeros((16,), jnp.float32)

        @pl.loop(0, _RPW, step=8)
        def _(r):
            pltpu.sync_copy(z_v, agg_sh.at[pl.ds(sid * _RPW + r, 8)])

        plsc.subcore_barrier()

        @pl.loop(0, _EPW, step=_KA)
        def _(j):
            pltpu.sync_copy(src_hbm.at[pl.ds(base + j, _KA)], sidx_v)
            pltpu.async_copy(h_hbm.at[sidx_v], rows_v, sem).wait()
            pltpu.sync_copy(dst_hbm.at[pl.ds(base + j, _KA)], didx_v)
            pltpu.sync_copy(rows_v, agg_sh.at[didx_v], add=True)

        plsc.subcore_barrier()
        pltpu.sync_copy(agg_sh.at[pl.ds(sid * _RPW, _RPW)],
                        out_hbm.at[cid, pl.ds(sid * _RPW, _RPW)])

    return k(h, src, dst)


def _tc_combine(aggp, lab3d, tim3d):
    def body(aggp_ref, lab_ref, tim_ref, agg_ref, gs_ref, gc_ref):
        i = pl.program_id(0)
        a = aggp_ref[0] + aggp_ref[1]
        agg_ref[...] = a

        @pl.when(i == 0)
        def _():
            gs_ref[...] = jnp.zeros_like(gs_ref)
            gc_ref[...] = jnp.zeros_like(gc_ref)

        g = tim_ref[0, 0, :] * _NL + lab_ref[0, 0, :]
        onehot = (lax.broadcasted_iota(jnp.int32, (_NL * _NT, _R), 0)
                  == g[None, :]).astype(jnp.float32)
        gs_ref[...] += lax.dot_general(onehot, a, (((1,), (0,)), ((), ())),
                                       preferred_element_type=jnp.float32)
        gc_ref[...] += jnp.sum(onehot, axis=1)[:, None]

    return pl.pallas_call(
        body,
        grid=(_GRID,),
        in_specs=[
            pl.BlockSpec((_NC, _R, _D), lambda i: (0, i, 0)),
            pl.BlockSpec((1, 1, _R), lambda i: (i, 0, 0)),
            pl.BlockSpec((1, 1, _R), lambda i: (i, 0, 0)),
        ],
        out_specs=[
            pl.BlockSpec((_R, _D), lambda i: (i, 0)),
            pl.BlockSpec((_NL * _NT, _D), lambda i: (0, 0)),
            pl.BlockSpec((_NL * _NT, _D), lambda i: (0, 0)),
        ],
        out_shape=[
            jax.ShapeDtypeStruct((_NPAD, _D), jnp.float32),
            jax.ShapeDtypeStruct((_NL * _NT, _D), jnp.float32),
            jax.ShapeDtypeStruct((_NL * _NT, _D), jnp.float32),
        ],
    )(aggp, lab3d, tim3d)


def _tc_transform(agg, lab3d, tim3d, norm, t_all, c_all):
    ng = _NL * _SPLIT

    def body(agg_ref, lab_ref, tim_ref, norm_ref, t_ref, c_ref, out_ref,
             cs_ref):
        i = pl.program_id(0)
        x = agg_ref[...]
        lab = lab_ref[0, 0, :]
        tim = tim_ref[0, 0, :]
        train = (tim < _SPLIT) & (lab >= 0)
        gid = jnp.where(train, lab * _SPLIT + tim, ng)
        acc = x * (~train).astype(jnp.float32)[:, None]
        for g in range(ng):
            m = (gid == g).astype(jnp.float32)[:, None]
            xm = x * m
            acc += lax.dot_general(xm, t_ref[g], (((1,), (1,)), ((), ())),
                                   preferred_element_type=jnp.float32)
            acc += m * c_ref[g][None, :]
        out = acc * norm_ref[...]
        out_ref[...] = out

        @pl.when(i == 0)
        def _():
            cs_ref[...] = jnp.zeros_like(cs_ref)

        cs_ref[0, :] += jnp.sum(out, axis=0)
        cs_ref[1, :] += jnp.sum(out * out, axis=0)

    return pl.pallas_call(
        body,
        grid=(_GRID,),
        in_specs=[
            pl.BlockSpec((_R, _D), lambda i: (i, 0)),
            pl.BlockSpec((1, 1, _R), lambda i: (i, 0, 0)),
            pl.BlockSpec((1, 1, _R), lambda i: (i, 0, 0)),
            pl.BlockSpec((_R, 1), lambda i: (i, 0)),
            pl.BlockSpec((ng, _D, _D), lambda i: (0, 0, 0)),
            pl.BlockSpec((ng, _D), lambda i: (0, 0)),
        ],
        out_specs=[
            pl.BlockSpec((_R, _D), lambda i: (i, 0)),
            pl.BlockSpec((2, _D), lambda i: (0, 0)),
        ],
        out_shape=[
            jax.ShapeDtypeStruct((_NPAD, _D), jnp.float32),
            jax.ShapeDtypeStruct((2, _D), jnp.float32),
        ],
    )(agg, lab3d, tim3d, norm, t_all, c_all)


def _tc_final(outv, ws, cvec):
    def body(o_ref, w_ref, c_ref, f_ref):
        f_ref[...] = lax.dot_general(
            o_ref[...], w_ref[...], (((1,), (1,)), ((), ())),
            preferred_element_type=jnp.float32) + c_ref[0][None, :]

    return pl.pallas_call(
        body,
        grid=(_GRID,),
        in_specs=[
            pl.BlockSpec((_R, _D), lambda i: (i, 0)),
            pl.BlockSpec((_D, _D), lambda i: (0, 0)),
            pl.BlockSpec((1, _D), lambda i: (0, 0)),
        ],
        out_specs=pl.BlockSpec((_R, _D), lambda i: (i, 0)),
        out_shape=jax.ShapeDtypeStruct((_NPAD, _D), jnp.float32),
    )(outv, ws, cvec)


def kernel(feat, edge_index, labels, times, P, W, b):
    src = edge_index[0]
    dst = edge_index[1]
    pad = _NPAD - _N
    lab_p = jnp.pad(labels, (0, pad), constant_values=-1)
    tim_p = jnp.pad(times, (0, pad), constant_values=127)
    lab3d = lab_p.reshape(_GRID, 1, _R)
    tim3d = tim_p.reshape(_GRID, 1, _R)

    degp = _sc_deg(dst)

    deg = jnp.clip(degp[0, :_N] + degp[1, :_N], 1.0, None)
    norm = (deg ** -0.5)[:, None]
    h = feat * norm
    prev_cov = []
    for y in range(_NL):
        m = (labels == y).astype(feat.dtype)[:, None]
        cnt = jnp.sum(m)
        mu = jnp.sum(h * m, axis=0) / cnt
        cen = (h - mu[None, :]) * m
        prev_cov.append(cen.T @ cen / (cnt - 1.0))

    cur = [[None] * _NT for _ in range(_NL)]
    for y1 in range(_NL):
        for t1 in range(_NT):
            denom = jnp.asarray(0.0, feat.dtype)
            for y2 in range(_NL):
                for t2 in range(_NT):
                    c = 2.0 if abs(t2 - t1) > min(_NT - 1 - t1, t1) else 1.0
                    denom = denom + P[y1, t1, y2, t2] * c
            denom = denom * denom
            acc = jnp.zeros((_D, _D), feat.dtype)
            for y2 in range(_NL):
                temp = jnp.asarray(0.0, feat.dtype)
                for t2 in range(_NT):
                    c = 4.0 if abs(t2 - t1) > min(_NT - 1 - t1, t1) else 1.0
                    temp = temp + P[y1, t1, y2, t2] * c
                temp = temp / denom
                acc = acc + temp * prev_cov[y2]
            cur[y1][t1] = acc

    stack = jnp.stack([cur[y][_NT - 1] for y in range(_NL)]
                      + [cur[y][t] for y in range(_NL)
                         for t in range(_SPLIT)])
    L_all, Q_all = jnp.linalg.eigh(stack)
    trans = [[None] * _SPLIT for _ in range(_NL)]
    for y1 in range(_NL):
        A = Q_all[y1] @ jnp.diag(jnp.sqrt(L_all[y1]))
        for t1 in range(_SPLIT):
            k = _NL + y1 * _SPLIT + t1
            trans[y1][t1] = A @ jnp.diag(1.0 / jnp.sqrt(L_all[k])) @ Q_all[k].T
    t_all = jnp.stack([trans[y][t] for y in range(_NL)
                       for t in range(_SPLIT)])

    h_p = jnp.pad(h, ((0, pad), (0, 0)))
    norm_p = jnp.pad(norm, ((0, pad), (0, 0)), constant_values=1.0)
    aggp = _sc_agg(h_p, src, dst)
    agg, gsum, gcnt = _tc_combine(aggp, lab3d, tim3d)

    mean = gsum / jnp.maximum(1.0, gcnt)
    mu_all = jnp.transpose(mean.reshape(_NT, _NL, _D),
                           (1, 0, 2))[:, :_SPLIT].reshape(_NL * _SPLIT, _D)
    c_all = mu_all - jnp.einsum('gd,ged->ge', mu_all, t_all)

    outv, cs = _tc_transform(agg, lab3d, tim3d, norm_p, t_all, c_all)


    m = cs[0] / float(_N)
    var = (cs[1] - float(_N) * m * m) / float(_N - 1)
    s = jnp.sqrt(var)
    ws = W / s[None, :]
    cvec = (b - (m / s) @ W.T)[None, :]

    final = _tc_final(outv, ws, cvec)
    return final[:_N]


# speedup vs baseline: 1.3137x; 1.1357x over previous
import functools

import jax
import jax.numpy as jnp
from jax import lax
from jax.experimental import pallas as pl
from jax.experimental.pallas import tpu as pltpu
from jax.experimental.pallas import tpu_sc as plsc

_N = 10000
_E = 320000
_D = 128
_NL = 4
_NT = 8
_SPLIT = 6

_NC, _NS = 2, 16
_NW = _NC * _NS
_EPW = _E // _NW
_KW = 80
_KA = 80
_NPAD = 10240
_RPW = _NPAD // _NS

_R = 256
_GRID = _NPAD // _R


def _sc_mesh():
    return plsc.VectorSubcoreMesh(core_axis_name="c", subcore_axis_name="s")


def _sc_deg(dst):
    @functools.partial(
        pl.kernel,
        mesh=_sc_mesh(),
        out_type=jax.ShapeDtypeStruct((_NC * _NPAD,), jnp.float32),
        scratch_types=[
            pltpu.VMEM((_KW,), jnp.int32),
            pltpu.VMEM((_KW,), jnp.float32),
            pltpu.VMEM((_RPW,), jnp.float32),
            pltpu.VMEM_SHARED((_NPAD,), jnp.float32),
        ],
    )
    def k(dst_hbm, out_hbm, idx_v, ones_v, z_v, deg_sh):
        cid = lax.axis_index("c")
        sid = lax.axis_index("s")
        base = (cid * _NS + sid) * _EPW

        @pl.loop(0, _KW, step=16)
        def _(i):
            ones_v[pl.ds(i, 16)] = jnp.full((16,), 1.0, jnp.float32)

        @pl.loop(0, _RPW, step=16)
        def _(i):
            z_v[pl.ds(i, 16)] = jnp.zeros((16,), jnp.float32)

        pltpu.sync_copy(z_v, deg_sh.at[pl.ds(sid * _RPW, _RPW)])
        plsc.subcore_barrier()

        @pl.loop(0, _EPW, step=_KW)
        def _(j):
            pltpu.sync_copy(dst_hbm.at[pl.ds(base + j, _KW)], idx_v)
            pltpu.sync_copy(ones_v, deg_sh.at[idx_v], add=True)

        plsc.subcore_barrier()
        pltpu.sync_copy(deg_sh.at[pl.ds(sid * _RPW, _RPW)], z_v)
        pltpu.sync_copy(z_v, out_hbm.at[pl.ds(cid * _NPAD + sid * _RPW, _RPW)])

    return k(dst).reshape(_NC, _NPAD)


def _sc_agg(h, src, dst):
    @functools.partial(
        pl.kernel,
        mesh=_sc_mesh(),
        out_type=jax.ShapeDtypeStruct((_NC, _NPAD, _D), jnp.float32),
        scratch_types=[
            pltpu.VMEM((_KA,), jnp.int32),
            pltpu.VMEM((_KA,), jnp.int32),
            pltpu.VMEM((_KA, _D), jnp.float32),
            pltpu.VMEM((8, _D), jnp.float32),
            pltpu.VMEM_SHARED((_NPAD, _D), jnp.float32),
            pltpu.SemaphoreType.DMA,
        ],
    )
    def k(h_hbm, src_hbm, dst_hbm, out_hbm, sidx_v, didx_v, rows_v, z_v,
          agg_sh, sem):
        cid = lax.axis_index("c")
        sid = lax.axis_index("s")
        base = (cid * _NS + sid) * _EPW

        @pl.loop(0, 8)
        def _(r):
            @pl.loop(0, _D, step=16)
            def _(i):
                z_v[r, pl.ds(i, 16)] = jnp.zeros((16,), jnp.float32)

        @pl.loop(0, _RPW, step=8)
        def _(r):
            pltpu.sync_copy(z_v, agg_sh.at[pl.ds(sid * _RPW + r, 8)])

        plsc.subcore_barrier()

        @pl.loop(0, _EPW, step=_KA)
        def _(j):
            pltpu.sync_copy(src_hbm.at[pl.ds(base + j, _KA)], sidx_v)
            pltpu.async_copy(h_hbm.at[sidx_v], rows_v, sem).wait()
            pltpu.sync_copy(dst_hbm.at[pl.ds(base + j, _KA)], didx_v)
            pltpu.sync_copy(rows_v, agg_sh.at[didx_v], add=True)

        plsc.subcore_barrier()
        pltpu.sync_copy(agg_sh.at[pl.ds(sid * _RPW, _RPW)],
                        out_hbm.at[cid, pl.ds(sid * _RPW, _RPW)])

    return k(h, src, dst)


def _tc_combine(aggp, lab3d, tim3d):
    def body(aggp_ref, lab_ref, tim_ref, agg_ref, gs_ref, gc_ref):
        i = pl.program_id(0)
        a = aggp_ref[0] + aggp_ref[1]
        agg_ref[...] = a

        @pl.when(i == 0)
        def _():
            gs_ref[...] = jnp.zeros_like(gs_ref)
            gc_ref[...] = jnp.zeros_like(gc_ref)

        g = tim_ref[0, 0, :] * _NL + lab_ref[0, 0, :]
        onehot = (lax.broadcasted_iota(jnp.int32, (_NL * _NT, _R), 0)
                  == g[None, :]).astype(jnp.float32)
        gs_ref[...] += lax.dot_general(onehot, a, (((1,), (0,)), ((), ())),
                                       preferred_element_type=jnp.float32)
        gc_ref[...] += jnp.sum(onehot, axis=1)[:, None]

    return pl.pallas_call(
        body,
        grid=(_GRID,),
        in_specs=[
            pl.BlockSpec((_NC, _R, _D), lambda i: (0, i, 0)),
            pl.BlockSpec((1, 1, _R), lambda i: (i, 0, 0)),
            pl.BlockSpec((1, 1, _R), lambda i: (i, 0, 0)),
        ],
        out_specs=[
            pl.BlockSpec((_R, _D), lambda i: (i, 0)),
            pl.BlockSpec((_NL * _NT, _D), lambda i: (0, 0)),
            pl.BlockSpec((_NL * _NT, _D), lambda i: (0, 0)),
        ],
        out_shape=[
            jax.ShapeDtypeStruct((_NPAD, _D), jnp.float32),
            jax.ShapeDtypeStruct((_NL * _NT, _D), jnp.float32),
            jax.ShapeDtypeStruct((_NL * _NT, _D), jnp.float32),
        ],
    )(aggp, lab3d, tim3d)


def _tc_transform(agg, lab3d, tim3d, norm, t_all, c_all):
    ng = _NL * _SPLIT

    def body(agg_ref, lab_ref, tim_ref, norm_ref, t_ref, c_ref, out_ref,
             cs_ref):
        i = pl.program_id(0)
        x = agg_ref[...]
        lab = lab_ref[0, 0, :]
        tim = tim_ref[0, 0, :]
        train = (tim < _SPLIT) & (lab >= 0)
        gid = jnp.where(train, lab * _SPLIT + tim, ng)
        acc = x * (~train).astype(jnp.float32)[:, None]
        for g in range(ng):
            m = (gid == g).astype(jnp.float32)[:, None]
            xm = x * m
            acc += lax.dot_general(xm, t_ref[g], (((1,), (1,)), ((), ())),
                                   preferred_element_type=jnp.float32)
            acc += m * c_ref[g][None, :]
        out = acc * norm_ref[...]
        out_ref[...] = out

        @pl.when(i == 0)
        def _():
            cs_ref[...] = jnp.zeros_like(cs_ref)

        cs_ref[0, :] += jnp.sum(out, axis=0)
        cs_ref[1, :] += jnp.sum(out * out, axis=0)

    return pl.pallas_call(
        body,
        grid=(_GRID,),
        in_specs=[
            pl.BlockSpec((_R, _D), lambda i: (i, 0)),
            pl.BlockSpec((1, 1, _R), lambda i: (i, 0, 0)),
            pl.BlockSpec((1, 1, _R), lambda i: (i, 0, 0)),
            pl.BlockSpec((_R, 1), lambda i: (i, 0)),
            pl.BlockSpec((ng, _D, _D), lambda i: (0, 0, 0)),
            pl.BlockSpec((ng, _D), lambda i: (0, 0)),
        ],
        out_specs=[
            pl.BlockSpec((_R, _D), lambda i: (i, 0)),
            pl.BlockSpec((2, _D), lambda i: (0, 0)),
        ],
        out_shape=[
            jax.ShapeDtypeStruct((_NPAD, _D), jnp.float32),
            jax.ShapeDtypeStruct((2, _D), jnp.float32),
        ],
    )(agg, lab3d, tim3d, norm, t_all, c_all)


def _tc_final(outv, ws, cvec):
    def body(o_ref, w_ref, c_ref, f_ref):
        f_ref[...] = lax.dot_general(
            o_ref[...], w_ref[...], (((1,), (1,)), ((), ())),
            preferred_element_type=jnp.float32) + c_ref[0][None, :]

    return pl.pallas_call(
        body,
        grid=(_GRID,),
        in_specs=[
            pl.BlockSpec((_R, _D), lambda i: (i, 0)),
            pl.BlockSpec((_D, _D), lambda i: (0, 0)),
            pl.BlockSpec((1, _D), lambda i: (0, 0)),
        ],
        out_specs=pl.BlockSpec((_R, _D), lambda i: (i, 0)),
        out_shape=jax.ShapeDtypeStruct((_NPAD, _D), jnp.float32),
    )(outv, ws, cvec)


def kernel(feat, edge_index, labels, times, P, W, b):
    src = edge_index[0]
    dst = edge_index[1]
    pad = _NPAD - _N
    lab_p = jnp.pad(labels, (0, pad), constant_values=-1)
    tim_p = jnp.pad(times, (0, pad), constant_values=127)
    lab3d = lab_p.reshape(_GRID, 1, _R)
    tim3d = tim_p.reshape(_GRID, 1, _R)

    degp = _sc_deg(dst)

    deg = jnp.clip(degp[0, :_N] + degp[1, :_N], 1.0, None)
    norm = (deg ** -0.5)[:, None]
    h = feat * norm
    prev_cov = []
    for y in range(_NL):
        m = (labels == y).astype(feat.dtype)[:, None]
        cnt = jnp.sum(m)
        mu = jnp.sum(h * m, axis=0) / cnt
        cen = (h - mu[None, :]) * m
        prev_cov.append(cen.T @ cen / (cnt - 1.0))

    c2m = [[2.0 if abs(t2 - t1) > min(_NT - 1 - t1, t1) else 1.0
            for t2 in range(_NT)] for t1 in range(_NT)]
    c4m = [[4.0 if abs(t2 - t1) > min(_NT - 1 - t1, t1) else 1.0
            for t2 in range(_NT)] for t1 in range(_NT)]
    c2f = jnp.asarray(c2m, feat.dtype)
    c4f = jnp.asarray(c4m, feat.dtype)
    Pc2 = P * c2f[None, :, None, :]
    Pc4 = P * c4f[None, :, None, :]
    den = jnp.zeros((_NL, _NT), feat.dtype)
    for y2 in range(_NL):
        for t2 in range(_NT):
            den = den + Pc2[:, :, y2, t2]
    den = den * den
    num = jnp.zeros((_NL, _NT, _NL), feat.dtype)
    for t2 in range(_NT):
        num = num + Pc4[:, :, :, t2]
    temp = num / den[:, :, None]
    pc_stack = jnp.stack(prev_cov)
    cur4 = jnp.zeros((_NL, _NT, _D, _D), feat.dtype)
    for y2 in range(_NL):
        cur4 = cur4 + temp[:, :, y2, None, None] * pc_stack[y2][None, None]
    cur = [[cur4[y1, t1] for t1 in range(_NT)] for y1 in range(_NL)]

    stack = jnp.stack([cur[y][_NT - 1] for y in range(_NL)]
                      + [cur[y][t] for y in range(_NL)
                         for t in range(_SPLIT)])
    L_all, Q_all = jnp.linalg.eigh(stack)
    trans = [[None] * _SPLIT for _ in range(_NL)]
    for y1 in range(_NL):
        A = Q_all[y1] @ jnp.diag(jnp.sqrt(L_all[y1]))
        for t1 in range(_SPLIT):
            k = _NL + y1 * _SPLIT + t1
            trans[y1][t1] = A @ jnp.diag(1.0 / jnp.sqrt(L_all[k])) @ Q_all[k].T
    t_all = jnp.stack([trans[y][t] for y in range(_NL)
                       for t in range(_SPLIT)])

    h_p = jnp.pad(h, ((0, pad), (0, 0)))
    norm_p = jnp.pad(norm, ((0, pad), (0, 0)), constant_values=1.0)
    aggp = _sc_agg(h_p, src, dst)
    agg, gsum, gcnt = _tc_combine(aggp, lab3d, tim3d)

    mean = gsum / jnp.maximum(1.0, gcnt)
    mu_all = jnp.transpose(mean.reshape(_NT, _NL, _D),
                           (1, 0, 2))[:, :_SPLIT].reshape(_NL * _SPLIT, _D)
    c_all = mu_all - jnp.einsum('gd,ged->ge', mu_all, t_all)

    outv, cs = _tc_transform(agg, lab3d, tim3d, norm_p, t_all, c_all)


    m = cs[0] / float(_N)
    var = (cs[1] - float(_N) * m * m) / float(_N - 1)
    s = jnp.sqrt(var)
    ws = W / s[None, :]
    cvec = (b - (m / s) @ W.T)[None, :]

    final = _tc_final(outv, ws, cvec)
    return final[:_N]
